# baseline (speedup 1.0000x reference)
import functools

import jax
import jax.numpy as jnp
from jax import lax
from jax.experimental import pallas as pl
from jax.experimental.pallas import tpu as pltpu, tpu_sc as plsc

B = 4096
D = 1024
KNB = 5
LOSS_W = 0.5

RB = 2048
NBLK = B // RB
SEGS = 2
SEG = B // SEGS
SEG_NBLK = SEG // RB

NC, NS, L = 2, 16, 16
NW = NC * NS
RW = SEG // NW
CH = 16
NCHUNK = RW // CH
PAIRS = CH * KNB
EGRP = -(-PAIRS // L)
D2 = D // 2
CSTEPS = D2 // L


def _norm_body(hc_ref, hp_ref, hcn_ref, hpnb_ref):
    x = hc_ref[...]
    n = jnp.sqrt(jnp.sum(x * x, axis=1, keepdims=True))
    xn = x / jnp.maximum(n, 1e-12)
    lo = lax.convert_element_type(
        lax.bitcast_convert_type(xn[:, :D2].astype(jnp.bfloat16),
                                 jnp.int16), jnp.int32)
    hi = lax.convert_element_type(
        lax.bitcast_convert_type(xn[:, D2:].astype(jnp.bfloat16),
                                 jnp.int16), jnp.int32)
    hcn_ref[...] = (lo & 0xFFFF) | (hi << 16)
    y = hp_ref[...]
    m = jnp.sqrt(jnp.sum(y * y, axis=1, keepdims=True))
    hpnb_ref[...] = (y / jnp.maximum(m, 1e-12)).astype(jnp.bfloat16)


_normalize = pl.pallas_call(
    _norm_body,
    grid=(NBLK,),
    in_specs=[pl.BlockSpec((RB, D), lambda i: (i, 0)),
              pl.BlockSpec((RB, D), lambda i: (i, 0))],
    out_specs=[pl.BlockSpec((RB, D2), lambda i: (i, 0)),
               pl.BlockSpec((RB, D), lambda i: (i, 0))],
    out_shape=[jax.ShapeDtypeStruct((B, D2), jnp.int32),
               jax.ShapeDtypeStruct((B, D), jnp.bfloat16)],
)


def _make_topk_body(seg):
  def _topk_body(blk_ref, all_ref, idx_ref):
    a = blk_ref[...]
    bm = all_ref[...]
    g = lax.dot_general(a, bm, (((1,), (1,)), ((), ())),
                        preferred_element_type=jnp.float32)
    col = lax.broadcasted_iota(jnp.int32, (RB, B), 1)
    rowid = (lax.broadcasted_iota(jnp.int32, (RB, B), 0)
             + (pl.program_id(0) + seg * SEG_NBLK) * RB)
    gi = lax.bitcast_convert_type(g, jnp.int32)
    neg = jnp.float32(-3.0e38)
    key = jnp.where(
        col == rowid, neg,
        lax.bitcast_convert_type((gi & ~0xFFF) | (0xFFF - col),
                                 jnp.float32))
    m = None
    for t in range(KNB):
        masked = key if t == 0 else jnp.where(key < m, key, neg)
        m = jnp.max(masked, axis=1, keepdims=True)
        mi = lax.bitcast_convert_type(m, jnp.int32)
        idx_ref[:, t:t + 1] = 0xFFF - (mi & 0xFFF)

  return _topk_body


@functools.cache
def _topk_seg(seg):
    return pl.pallas_call(
        _make_topk_body(seg),
        grid=(SEG_NBLK,),
        in_specs=[pl.BlockSpec((RB, D), lambda i: (i + seg * SEG_NBLK, 0)),
                  pl.BlockSpec((B, D), lambda i: (0, 0))],
        out_specs=pl.BlockSpec((RB, 128), lambda i: (i, 0)),
        out_shape=jax.ShapeDtypeStruct((SEG, 128), jnp.int32),
    )


def _make_sc_loss_body(seg):
  def _sc_loss_body(hc_hbm, nbr_hbm, lab_hbm, out_hbm,
                    idx_v, rows_a, rows_b, hci_a, hci_b,
                    lab_v, e_v, out_v, sem_a, sem_b):
    wid = lax.axis_index("s") * NC + lax.axis_index("c")
    pltpu.sync_copy(lab_hbm, lab_v)
    pltpu.sync_copy(nbr_hbm.at[pl.ds(wid * RW * KNB, RW * KNB)], idx_v)

    bufs = ((rows_a, hci_a, sem_a), (rows_b, hci_b, sem_b))

    def issue(ch, buf):
        rows_v, hci_v, sem = buf
        loc = wid * RW + ch * CH
        return (pltpu.async_copy(
                    hc_hbm.at[idx_v.at[pl.ds(ch * PAIRS, PAIRS)]],
                    rows_v, sem),
                pltpu.async_copy(hc_hbm.at[pl.ds(seg * SEG + loc, CH)],
                                 hci_v, sem))

    pending = {0: issue(0, bufs[0]), 1: None}

    base0 = seg * SEG + wid * RW
    for gidx in range(EGRP * NCHUNK):
        pos = jnp.minimum(jnp.arange(L, dtype=jnp.int32) + gidx * L,
                          RW * KNB - 1)
        idxg = plsc.load_gather(idx_v, [pos])
        labj = plsc.load_gather(lab_v, [idxg])
        labi = plsc.load_gather(lab_v, [base0 + pos // KNB])
        e_v[pl.ds(gidx * L, L)] = jnp.where(labj == labi, 1.0, -1.0)

    def compute_chunk(ch, buf, total):
        rows_v, hci_v, _ = buf

        def i_body(i, tot):
            z = jnp.zeros((L,), jnp.float32)

            himask = jnp.int32(-65536)

            def bf2(w):
                return (plsc.bitcast(w << 16, jnp.float32),
                        plsc.bitcast(w & himask, jnp.float32))

            def c_body(c, accs):
                off = c * L
                a0, a1 = bf2(hci_v[i, pl.ds(off, L)])
                out = []
                for t in range(KNB):
                    b0, b1 = bf2(rows_v[i * KNB + t, pl.ds(off, L)])
                    out.append(accs[t] + a0 * b0 + a1 * b1)
                return tuple(out)

            accs = lax.fori_loop(0, CSTEPS, c_body, (z,) * KNB, unroll=4)
            e16 = plsc.load_gather(
                e_v, [(ch * CH + i) * KNB + jnp.arange(L, dtype=jnp.int32)])
            for t in range(KNB):
                dot = jnp.sum(accs[t])
                dist = jnp.maximum(2.0 - 2.0 * dot, 0.0)
                tot = tot + e16[t] * dist
            return tot

        return lax.fori_loop(0, CH, i_body, total)

    total = jnp.float32(0.0)
    for ch in range(NCHUNK):
        p = ch % 2
        for cp in pending[p]:
            cp.wait()
        if ch + 1 < NCHUNK:
            pending[1 - p] = issue(ch + 1, bufs[1 - p])
        total = compute_chunk(ch, bufs[p], total)

    lanes = jnp.arange(L, dtype=jnp.int32)
    out_v[...] = jnp.where(lanes == 0, total, 0.0)
    pltpu.sync_copy(out_v, out_hbm.at[wid])

  return _sc_loss_body


@functools.cache
def _sc_loss(seg):
    return pl.kernel(
        _make_sc_loss_body(seg),
        out_type=jax.ShapeDtypeStruct((NW, L), jnp.float32),
        compiler_params=pltpu.CompilerParams(needs_layout_passes=False),
        mesh=plsc.VectorSubcoreMesh(core_axis_name="c", subcore_axis_name="s",
                                    num_cores=NC, num_subcores=NS),
        scratch_types=[
            pltpu.VMEM((RW * KNB,), jnp.int32),
            pltpu.VMEM((PAIRS, D2), jnp.int32),
            pltpu.VMEM((PAIRS, D2), jnp.int32),
            pltpu.VMEM((CH, D2), jnp.int32),
            pltpu.VMEM((CH, D2), jnp.int32),
            pltpu.VMEM((B,), jnp.int32),
            pltpu.VMEM((RW * KNB + L,), jnp.float32),
            pltpu.VMEM((L,), jnp.float32),
            pltpu.SemaphoreType.DMA,
            pltpu.SemaphoreType.DMA,
        ],
    )


def kernel(hidden_current, hidden_previous, labels_current, labels_previous):
    hc32, hpn = _normalize(hidden_current.astype(jnp.float32),
                           lax.stop_gradient(hidden_previous).astype(jnp.float32))
    lab = labels_previous.astype(jnp.int32)
    total = jnp.float32(0.0)
    for seg in range(SEGS):
        nbr = _topk_seg(seg)(hpn, hpn)[:, :KNB]
        parts = _sc_loss(seg)(hc32, nbr.reshape(-1), lab)
        total = total + jnp.sum(parts)
    return (LOSS_W / (B * B)) * total


# speedup vs baseline: 1.0382x; 1.0382x over previous
import functools

import jax
import jax.numpy as jnp
from jax import lax
from jax.experimental import pallas as pl
from jax.experimental.pallas import tpu as pltpu, tpu_sc as plsc

B = 4096
D = 1024
KNB = 5
LOSS_W = 0.5

RB = 1024
NBLK = B // RB
SEGS = 2
SEG = B // SEGS
SEG_NBLK = SEG // RB

NC, NS, L = 2, 16, 16
NW = NC * NS
RW = SEG // NW
CH = 16
NCHUNK = RW // CH
PAIRS = CH * KNB
EGRP = -(-PAIRS // L)
D2 = D // 2
CSTEPS = D2 // L


def _norm_body(hc_ref, hp_ref, hcn_ref, hpnb_ref):
    x = hc_ref[...]
    n = jnp.sqrt(jnp.sum(x * x, axis=1, keepdims=True))
    xn = x / jnp.maximum(n, 1e-12)
    lo = lax.convert_element_type(
        lax.bitcast_convert_type(xn[:, :D2].astype(jnp.bfloat16),
                                 jnp.int16), jnp.int32)
    hi = lax.convert_element_type(
        lax.bitcast_convert_type(xn[:, D2:].astype(jnp.bfloat16),
                                 jnp.int16), jnp.int32)
    hcn_ref[...] = (lo & 0xFFFF) | (hi << 16)
    y = hp_ref[...]
    m = jnp.sqrt(jnp.sum(y * y, axis=1, keepdims=True))
    hpnb_ref[...] = (y / jnp.maximum(m, 1e-12)).astype(jnp.bfloat16)


_normalize = pl.pallas_call(
    _norm_body,
    grid=(NBLK,),
    in_specs=[pl.BlockSpec((RB, D), lambda i: (i, 0)),
              pl.BlockSpec((RB, D), lambda i: (i, 0))],
    out_specs=[pl.BlockSpec((RB, D2), lambda i: (i, 0)),
               pl.BlockSpec((RB, D), lambda i: (i, 0))],
    out_shape=[jax.ShapeDtypeStruct((B, D2), jnp.int32),
               jax.ShapeDtypeStruct((B, D), jnp.bfloat16)],
)


def _make_topk_body(seg):
  def _topk_body(blk_ref, all_ref, idx_ref):
    a = blk_ref[...]
    bm = all_ref[...]
    g = lax.dot_general(a, bm, (((1,), (1,)), ((), ())),
                        preferred_element_type=jnp.float32)
    col = lax.broadcasted_iota(jnp.int32, (RB, B), 1)
    rowid = (lax.broadcasted_iota(jnp.int32, (RB, B), 0)
             + (pl.program_id(0) + seg * SEG_NBLK) * RB)
    gi = lax.bitcast_convert_type(g, jnp.int32)
    neg = jnp.float32(-3.0e38)
    key = jnp.where(
        col == rowid, neg,
        lax.bitcast_convert_type((gi & ~0xFFF) | (0xFFF - col),
                                 jnp.float32))
    m = None
    for t in range(KNB):
        masked = key if t == 0 else jnp.where(key < m, key, neg)
        m = jnp.max(masked, axis=1, keepdims=True)
        mi = lax.bitcast_convert_type(m, jnp.int32)
        idx_ref[:, t:t + 1] = 0xFFF - (mi & 0xFFF)

  return _topk_body


@functools.cache
def _topk_seg(seg):
    return pl.pallas_call(
        _make_topk_body(seg),
        grid=(SEG_NBLK,),
        in_specs=[pl.BlockSpec((RB, D), lambda i: (i + seg * SEG_NBLK, 0)),
                  pl.BlockSpec((B, D), lambda i: (0, 0))],
        out_specs=pl.BlockSpec((RB, 128), lambda i: (i, 0)),
        out_shape=jax.ShapeDtypeStruct((SEG, 128), jnp.int32),
    )


def _make_sc_loss_body(seg):
  def _sc_loss_body(hc_hbm, nbr_hbm, lab_hbm, out_hbm,
                    idx_v, rows_a, rows_b, hci_a, hci_b,
                    lab_v, e_v, out_v, sem_a, sem_b):
    wid = lax.axis_index("s") * NC + lax.axis_index("c")
    pltpu.sync_copy(lab_hbm, lab_v)
    pltpu.sync_copy(nbr_hbm.at[pl.ds(wid * RW * KNB, RW * KNB)], idx_v)

    bufs = ((rows_a, hci_a, sem_a), (rows_b, hci_b, sem_b))

    def issue(ch, buf):
        rows_v, hci_v, sem = buf
        loc = wid * RW + ch * CH
        return (pltpu.async_copy(
                    hc_hbm.at[idx_v.at[pl.ds(ch * PAIRS, PAIRS)]],
                    rows_v, sem),
                pltpu.async_copy(hc_hbm.at[pl.ds(seg * SEG + loc, CH)],
                                 hci_v, sem))

    pending = {0: issue(0, bufs[0]), 1: None}

    base0 = seg * SEG + wid * RW
    for gidx in range(EGRP * NCHUNK):
        pos = jnp.minimum(jnp.arange(L, dtype=jnp.int32) + gidx * L,
                          RW * KNB - 1)
        idxg = plsc.load_gather(idx_v, [pos])
        labj = plsc.load_gather(lab_v, [idxg])
        labi = plsc.load_gather(lab_v, [base0 + pos // KNB])
        e_v[pl.ds(gidx * L, L)] = jnp.where(labj == labi, 1.0, -1.0)

    def compute_chunk(ch, buf, total):
        rows_v, hci_v, _ = buf

        def i_body(i, tot):
            z = jnp.zeros((L,), jnp.float32)

            himask = jnp.int32(-65536)

            def bf2(w):
                return (plsc.bitcast(w << 16, jnp.float32),
                        plsc.bitcast(w & himask, jnp.float32))

            def c_body(c, accs):
                off = c * L
                a0, a1 = bf2(hci_v[i, pl.ds(off, L)])
                out = []
                for t in range(KNB):
                    b0, b1 = bf2(rows_v[i * KNB + t, pl.ds(off, L)])
                    out.append(accs[t] + a0 * b0 + a1 * b1)
                return tuple(out)

            accs = lax.fori_loop(0, CSTEPS, c_body, (z,) * KNB, unroll=4)
            e16 = plsc.load_gather(
                e_v, [(ch * CH + i) * KNB + jnp.arange(L, dtype=jnp.int32)])
            for t in range(KNB):
                dot = jnp.sum(accs[t])
                dist = jnp.maximum(2.0 - 2.0 * dot, 0.0)
                tot = tot + e16[t] * dist
            return tot

        return lax.fori_loop(0, CH, i_body, total)

    total = jnp.float32(0.0)
    for ch in range(NCHUNK):
        p = ch % 2
        for cp in pending[p]:
            cp.wait()
        if ch + 1 < NCHUNK:
            pending[1 - p] = issue(ch + 1, bufs[1 - p])
        total = compute_chunk(ch, bufs[p], total)

    lanes = jnp.arange(L, dtype=jnp.int32)
    out_v[...] = jnp.where(lanes == 0, total, 0.0)
    pltpu.sync_copy(out_v, out_hbm.at[wid])

  return _sc_loss_body


@functools.cache
def _sc_loss(seg):
    return pl.kernel(
        _make_sc_loss_body(seg),
        out_type=jax.ShapeDtypeStruct((NW, L), jnp.float32),
        compiler_params=pltpu.CompilerParams(needs_layout_passes=False),
        mesh=plsc.VectorSubcoreMesh(core_axis_name="c", subcore_axis_name="s",
                                    num_cores=NC, num_subcores=NS),
        scratch_types=[
            pltpu.VMEM((RW * KNB,), jnp.int32),
            pltpu.VMEM((PAIRS, D2), jnp.int32),
            pltpu.VMEM((PAIRS, D2), jnp.int32),
            pltpu.VMEM((CH, D2), jnp.int32),
            pltpu.VMEM((CH, D2), jnp.int32),
            pltpu.VMEM((B,), jnp.int32),
            pltpu.VMEM((RW * KNB + L,), jnp.float32),
            pltpu.VMEM((L,), jnp.float32),
            pltpu.SemaphoreType.DMA,
            pltpu.SemaphoreType.DMA,
        ],
    )


def kernel(hidden_current, hidden_previous, labels_current, labels_previous):
    hc32, hpn = _normalize(hidden_current.astype(jnp.float32),
                           lax.stop_gradient(hidden_previous).astype(jnp.float32))
    lab = labels_previous.astype(jnp.int32)
    total = jnp.float32(0.0)
    for seg in range(SEGS):
        nbr = _topk_seg(seg)(hpn, hpn)[:, :KNB]
        parts = _sc_loss(seg)(hc32, nbr.reshape(-1), lab)
        total = total + jnp.sum(parts)
    return (LOSS_W / (B * B)) * total
